# baseline (speedup 1.0000x reference)
import jax
import jax.numpy as jnp
from jax.experimental import pallas as pl
from jax.experimental.pallas import tpu as pltpu

BB = 256


def _band_w1(conv1_w):
    dj = jnp.arange(5)
    c = jnp.arange(28)[:, None, None, None]
    p = jnp.arange(2)[None, :, None, None]
    jp = jnp.arange(12)[None, None, :, None]
    s = (dj[None, None, None, :] == c - 2 * jp - p).astype(jnp.float32)
    w = conv1_w[:, 0, :, :]
    vals = jnp.einsum('oxd,cpjd->xcpoj', w, s)
    return vals.reshape(140, 384)


def _band_w2(conv2_w):
    dj = jnp.arange(3)
    c = jnp.arange(12)[:, None, None]
    j2 = jnp.arange(10)[None, :, None]
    s2 = (dj[None, None, :] == c - j2).astype(jnp.float32)
    blocks = [
        jnp.einsum('oid,cjd->icoj', conv2_w[:, :, di, :], s2).reshape(192, 320)
        for di in range(3)
    ]
    return jnp.concatenate(blocks, axis=1)


def _fused_body(x_hbm, w1_ref, b1_ref, w2_ref, b2_ref, e1_ref, eb1_ref,
                e2_ref, eb2_ref, sw_ref, sb_ref, out_ref, xbuf, sems):
    i = pl.program_id(0)
    n = pl.num_programs(0)
    slot = jax.lax.rem(i, 2)
    nslot = jax.lax.rem(i + 1, 2)

    def xcopy(s, blk):
        return pltpu.make_async_copy(
            x_hbm.at[pl.ds(blk * BB, BB)], xbuf.at[s], sems.at[s])

    @pl.when(i == 0)
    def _():
        xcopy(0, 0).start()

    @pl.when(i + 1 < n)
    def _():
        xcopy(nslot, i + 1).start()

    xcopy(slot, i).wait()

    x5 = jnp.concatenate(
        [xbuf[slot, :, 0, di:di + 24, :] for di in range(5)], axis=2)
    x5m = x5.reshape(BB * 24, 140)
    h1 = jnp.dot(x5m, w1_ref[...], preferred_element_type=jnp.float32)

    m1 = jnp.maximum(h1[:, 0:192], h1[:, 192:384]).reshape(BB, 24, 192)
    mr = jnp.maximum(m1[:, 0:23, :], m1[:, 1:24, :])
    pooled = jnp.concatenate(
        [mr[:, 2 * rp:2 * rp + 1, :] for rp in range(12)]
        + [jnp.zeros((BB, 4, 192), jnp.float32)], axis=1)
    pooled = jnp.maximum(pooled + b1_ref[...][None, None, :], 0.0)

    y2 = jnp.dot(pooled.reshape(BB * 16, 192), w2_ref[...],
                 preferred_element_type=jnp.float32).reshape(BB, 16, 960)
    acc2 = (y2[:, 0:10, 0:320] + y2[:, 1:11, 320:640] + y2[:, 2:12, 640:960]
            + b2_ref[...][None, None, :])
    h2 = jnp.maximum(acc2, 0.0)

    acc = jnp.zeros((BB, 256), jnp.float32)
    for i2 in range(10):
        acc = acc + jnp.dot(h2[:, i2, :], e1_ref[i2],
                            preferred_element_type=jnp.float32)
    hid = jnp.maximum(acc + eb1_ref[...][None, :], 0.0)
    mid = jnp.maximum(jnp.dot(hid, e2_ref[...],
                              preferred_element_type=jnp.float32)
                      + eb2_ref[...][None, :], 0.0)
    logits = jnp.dot(mid, sw_ref[...],
                     preferred_element_type=jnp.float32) + sb_ref[...][None, :]

    m = jnp.max(logits, axis=-1, keepdims=True)
    e = jnp.exp(logits - m)
    out_ref[...] = e / jnp.sum(e, axis=-1, keepdims=True)


@jax.jit
def kernel(x, conv1_w, conv1_b, conv2_w, conv2_b, gate_w, gate_b,
           e_w1, e_b1, e_w2, e_b2, sm_w, sm_b):
    del gate_w, gate_b
    b = x.shape[0]
    w1band = _band_w1(conv1_w)
    w2band = _band_w2(conv2_w)
    b1rep = jnp.repeat(conv1_b, 12)
    b2rep = jnp.repeat(conv2_b, 10)
    e1p = e_w1.reshape(32, 10, 10, 256).transpose(1, 0, 2, 3).reshape(10, 320, 256)

    grid = (b // BB,)
    wspec = lambda shape: pl.BlockSpec(shape, lambda i: (0,) * len(shape))
    out = pl.pallas_call(
        _fused_body,
        grid=grid,
        in_specs=[
            pl.BlockSpec(memory_space=pl.ANY),
            wspec((140, 384)),
            wspec((192,)),
            wspec((192, 960)),
            wspec((320,)),
            wspec((10, 320, 256)),
            wspec((256,)),
            wspec((256, 128)),
            wspec((128,)),
            wspec((128, 10)),
            wspec((10,)),
        ],
        out_specs=pl.BlockSpec((BB, 10), lambda i: (i, 0)),
        out_shape=jax.ShapeDtypeStruct((b, 10), jnp.float32),
        scratch_shapes=[
            pltpu.VMEM((2, BB, 1, 28, 28), jnp.float32),
            pltpu.SemaphoreType.DMA((2,)),
        ],
        compiler_params=pltpu.CompilerParams(
            dimension_semantics=("arbitrary",)),
    )(x, w1band, b1rep, w2band, b2rep, e1p, e_b1, e_w2, e_b2, sm_w, sm_b)
    return out


# speedup vs baseline: 1.0034x; 1.0034x over previous
import jax
import jax.numpy as jnp
from jax.experimental import pallas as pl
from jax.experimental.pallas import tpu as pltpu

BB = 256


def _band_w1(conv1_w):
    dj = jnp.arange(5)
    c = jnp.arange(28)[:, None, None, None]
    p = jnp.arange(2)[None, :, None, None]
    jp = jnp.arange(12)[None, None, :, None]
    s = (dj[None, None, None, :] == c - 2 * jp - p).astype(jnp.float32)
    w = conv1_w[:, 0, :, :]
    vals = jnp.einsum('oxd,cpjd->xcpoj', w, s)
    return vals.reshape(140, 384)


def _band_w2(conv2_w):
    dj = jnp.arange(3)
    c = jnp.arange(12)[:, None, None]
    j2 = jnp.arange(10)[None, :, None]
    s2 = (dj[None, None, :] == c - j2).astype(jnp.float32)
    blocks = [
        jnp.einsum('oid,cjd->icoj', conv2_w[:, :, di, :], s2).reshape(192, 320)
        for di in range(3)
    ]
    return jnp.concatenate(blocks, axis=1)


def _fused_body(x_ref, w1_ref, b1_ref, w2_ref, b2_ref, e1_ref, eb1_ref,
                e2_ref, eb2_ref, sw_ref, sb_ref, out_ref):
    xb = x_ref[:, 0]

    x5 = jnp.concatenate([xb[:, di:di + 24, :] for di in range(5)], axis=2)
    x5m = x5.reshape(BB * 24, 140)
    h1 = jnp.dot(x5m, w1_ref[...], preferred_element_type=jnp.float32)

    m1 = jnp.maximum(h1[:, 0:192], h1[:, 192:384]).reshape(BB, 24, 192)
    mr = jnp.maximum(m1[:, 0:23, :], m1[:, 1:24, :])
    pooled = jnp.concatenate(
        [mr[:, 2 * rp:2 * rp + 1, :] for rp in range(12)]
        + [jnp.zeros((BB, 4, 192), jnp.float32)], axis=1)
    pooled = jnp.maximum(pooled + b1_ref[...][None, None, :], 0.0)

    y2 = jnp.dot(pooled.reshape(BB * 16, 192), w2_ref[...],
                 preferred_element_type=jnp.float32).reshape(BB, 16, 960)
    acc2 = (y2[:, 0:10, 0:320] + y2[:, 1:11, 320:640] + y2[:, 2:12, 640:960]
            + b2_ref[...][None, None, :])
    h2 = jnp.maximum(acc2, 0.0)

    acc = jnp.zeros((BB, 256), jnp.float32)
    for i2 in range(10):
        acc = acc + jnp.dot(h2[:, i2, :], e1_ref[i2],
                            preferred_element_type=jnp.float32)
    hid = jnp.maximum(acc + eb1_ref[...][None, :], 0.0)
    mid = jnp.maximum(jnp.dot(hid, e2_ref[...],
                              preferred_element_type=jnp.float32)
                      + eb2_ref[...][None, :], 0.0)
    logits = jnp.dot(mid, sw_ref[...],
                     preferred_element_type=jnp.float32) + sb_ref[...][None, :]

    m = jnp.max(logits, axis=-1, keepdims=True)
    e = jnp.exp(logits - m)
    out_ref[...] = e / jnp.sum(e, axis=-1, keepdims=True)


@jax.jit
def kernel(x, conv1_w, conv1_b, conv2_w, conv2_b, gate_w, gate_b,
           e_w1, e_b1, e_w2, e_b2, sm_w, sm_b):
    del gate_w, gate_b
    b = x.shape[0]
    w1band = _band_w1(conv1_w)
    w2band = _band_w2(conv2_w)
    b1rep = jnp.repeat(conv1_b, 12)
    b2rep = jnp.repeat(conv2_b, 10)
    e1p = e_w1.reshape(32, 10, 10, 256).transpose(1, 0, 2, 3).reshape(10, 320, 256)

    grid = (b // BB,)
    wspec = lambda shape: pl.BlockSpec(shape, lambda i: (0,) * len(shape))
    out = pl.pallas_call(
        _fused_body,
        grid=grid,
        in_specs=[
            pl.BlockSpec((BB, 1, 28, 28), lambda i: (i, 0, 0, 0)),
            wspec((140, 384)),
            wspec((192,)),
            wspec((192, 960)),
            wspec((320,)),
            wspec((10, 320, 256)),
            wspec((256,)),
            wspec((256, 128)),
            wspec((128,)),
            wspec((128, 10)),
            wspec((10,)),
        ],
        out_specs=pl.BlockSpec((BB, 10), lambda i: (i, 0)),
        out_shape=jax.ShapeDtypeStruct((b, 10), jnp.float32),
        compiler_params=pltpu.CompilerParams(
            dimension_semantics=("parallel",)),
    )(x, w1band, b1rep, w2band, b2rep, e1p, e_b1, e_w2, e_b2, sm_w, sm_b)
    return out
